# baseline (speedup 1.0000x reference)
import functools

import jax
import jax.numpy as jnp
from jax import lax
from jax.experimental import pallas as pl
from jax.experimental.pallas import tpu as pltpu
from jax.experimental.pallas import tpu_sc as plsc

N, NC, C = 2500, 6, 128
HEADS, LEVELS, POINTS = 8, 3, 8
LP = LEVELS * POINTS
NP = 2560
HD = C // HEADS
SPATIAL = ((32, 88), (16, 44), (8, 22))
LVL_BASE = (0, 2816, 3520)
PIX = 3696
SLAB = PIX * HD

QHALF = NP // 2
CHUNK = 128
NGROUP = CHUNK // 16
NCH = QHALF // CHUNK


def _value_kernel(ff_ref, wval_ref, bval_ref, cam_ref, lvl_ref, out_ref):
    ff = ff_ref[0]
    res = lax.dot_general(ff, wval_ref[...], (((1,), (1,)), ((), ())),
                          preferred_element_type=jnp.float32)
    res = res + bval_ref[0:1, :]
    emb = cam_ref[0] + lvl_ref[...]
    bias3 = lax.dot_general(emb, wval_ref[...], (((1,), (1,)), ((), ())),
                            preferred_element_type=jnp.float32)
    for h in range(HEADS):
        c0 = h * HD
        for l in range(LEVELS):
            r0 = LVL_BASE[l]
            r1 = LVL_BASE[l + 1] if l + 1 < LEVELS else PIX
            out_ref[0, h, r0:r1, :] = (res[r0:r1, c0:c0 + HD]
                                       + bias3[l:l + 1, c0:c0 + HD])


def _project_value(ffT, W_val, b_val, cams_embeds, level_embeds):
    bval_m = jnp.broadcast_to(b_val[None, :], (8, C))
    return pl.pallas_call(
        _value_kernel,
        grid=(NC,),
        in_specs=[
            pl.BlockSpec((1, PIX, C), lambda i: (i, 0, 0)),
            pl.BlockSpec((C, C), lambda i: (0, 0)),
            pl.BlockSpec((8, C), lambda i: (0, 0)),
            pl.BlockSpec((1, 1, C), lambda i: (i, 0, 0)),
            pl.BlockSpec((LEVELS, C), lambda i: (0, 0)),
        ],
        out_specs=pl.BlockSpec((1, HEADS, PIX, HD), lambda i: (i, 0, 0, 0)),
        out_shape=jax.ShapeDtypeStruct((NC, HEADS, PIX, HD), jnp.float32),
    )(ffT, W_val, bval_m, cams_embeds[:, None, :], level_embeds)


_QB = 512


def _offattn_kernel(ft_ref, woff_ref, boff_ref, wattn_ref, battn_ref,
                    so_ref, aw_ref):
    ft = ft_ref[...]
    lo = lax.dot_general(woff_ref[...], ft, (((1,), (0,)), ((), ())),
                         preferred_element_type=jnp.float32)
    lo = lo + boff_ref[:, 0:1]
    so_ref[...] = lo.reshape(2, HEADS * LP, _QB)
    la = lax.dot_general(wattn_ref[...], ft, (((1,), (0,)), ((), ())),
                         preferred_element_type=jnp.float32)
    la = la + battn_ref[:, 0:1]
    for h in range(HEADS):
        s = la[h * LP:(h + 1) * LP, :]
        m = jnp.max(s, axis=0, keepdims=True)
        e = jnp.exp(s - m)
        aw_ref[h * LP:(h + 1) * LP, :] = e / jnp.sum(e, axis=0, keepdims=True)


def _off_attn(featT, W_off2, b_off2, W_attn, b_attn):
    boff_m = jnp.broadcast_to(b_off2[:, None], (2 * HEADS * LP, 8))
    battn_m = jnp.broadcast_to(b_attn[:, None], (HEADS * LP, 8))
    return pl.pallas_call(
        _offattn_kernel,
        grid=(NP // _QB,),
        in_specs=[
            pl.BlockSpec((C, _QB), lambda i: (0, i)),
            pl.BlockSpec((2 * HEADS * LP, C), lambda i: (0, 0)),
            pl.BlockSpec((2 * HEADS * LP, 8), lambda i: (0, 0)),
            pl.BlockSpec((HEADS * LP, C), lambda i: (0, 0)),
            pl.BlockSpec((HEADS * LP, 8), lambda i: (0, 0)),
        ],
        out_specs=[
            pl.BlockSpec((2, HEADS * LP, _QB), lambda i: (0, 0, i)),
            pl.BlockSpec((HEADS * LP, _QB), lambda i: (0, i)),
        ],
        out_shape=[
            jax.ShapeDtypeStruct((2, HEADS * LP, NP), jnp.float32),
            jax.ShapeDtypeStruct((HEADS * LP, NP), jnp.float32),
        ],
    )(featT, W_off2, boff_m, W_attn, battn_m)


def _sc_sample_body(value_hbm, so_hbm, aw_hbm, coor_hbm, out_hbm,
                    slab_v, sox_v, soy_v, aw_v, cx_v, cy_v, out_v,
                    insem, outsem):
    info = plsc.get_sparse_core_info()
    ncores = info.num_cores
    wid = lax.axis_index("s") * ncores + lax.axis_index("c")

    def unit_body(j, _):
        u = wid * 3 + j
        cam = u // 16
        r = u - cam * 16
        h = r // 2
        qh = r - h * 2
        pltpu.sync_copy(value_hbm.at[pl.ds((cam * HEADS + h) * SLAB, SLAB)],
                        slab_v)
        qs = qh * QHALF

        def in_copies(scn):
            buf = scn & 1
            qb = qs + scn * CHUNK
            return (
                pltpu.make_async_copy(
                    so_hbm.at[0, pl.ds(h * LP, LP), pl.ds(qb, CHUNK)],
                    sox_v.at[buf], insem),
                pltpu.make_async_copy(
                    so_hbm.at[1, pl.ds(h * LP, LP), pl.ds(qb, CHUNK)],
                    soy_v.at[buf], insem),
                pltpu.make_async_copy(
                    aw_hbm.at[pl.ds(h * LP, LP), pl.ds(qb, CHUNK)],
                    aw_v.at[buf], insem),
                pltpu.make_async_copy(
                    coor_hbm.at[pl.ds(cam * NP + qb, CHUNK)],
                    cx_v.at[buf], insem),
                pltpu.make_async_copy(
                    coor_hbm.at[pl.ds((NC + cam) * NP + qb, CHUNK)],
                    cy_v.at[buf], insem),
            )

        def out_copy(scn):
            buf = scn & 1
            qb = qs + scn * CHUNK
            return pltpu.make_async_copy(
                out_v.at[buf],
                out_hbm.at[cam, h, pl.ds(qb, CHUNK), :], outsem)

        for c in in_copies(0):
            c.start()

        def chunk_body(scn, _2):
            buf = scn & 1
            qb = qs + scn * CHUNK
            for c in in_copies(scn):
                c.wait()

            @pl.when(scn + 1 < NCH)
            def _fire_next():
                for c in in_copies(scn + 1):
                    c.start()

            @pl.when(scn >= 2)
            def _reclaim_out():
                out_copy(scn - 2).wait()

            @plsc.parallel_loop(0, NGROUP)
            def group_body(g):
                qoff = g * 16
                cx = cx_v[buf, pl.ds(qoff, 16)]
                cy = cy_v[buf, pl.ds(qoff, 16)]
                acc = [jnp.zeros((16,), jnp.float32) for _ in range(16)]
                for l in range(LEVELS):
                    hl, wl = SPATIAL[l]
                    base = LVL_BASE[l]
                    xb = cx * float(wl) - 0.5
                    yb = cy * float(hl) - 0.5

                    def p_body(p, acc_c, l=l, hl=hl, wl=wl, base=base,
                               xb=xb, yb=yb):
                        lp = l * POINTS + p
                        sx = sox_v[buf, lp, pl.ds(qoff, 16)]
                        sy = soy_v[buf, lp, pl.ds(qoff, 16)]
                        a = aw_v[buf, lp, pl.ds(qoff, 16)]
                        x = xb + sx
                        y = yb + sy
                        xt = x.astype(jnp.int32)
                        xtf = xt.astype(jnp.float32)
                        x0f = jnp.where(xtf > x, xtf - 1.0, xtf)
                        x0 = x0f.astype(jnp.int32)
                        yt = y.astype(jnp.int32)
                        ytf = yt.astype(jnp.float32)
                        y0f = jnp.where(ytf > y, ytf - 1.0, ytf)
                        y0 = y0f.astype(jnp.int32)
                        fx = x - x0f
                        fy = y - y0f
                        x1 = x0 + 1
                        y1 = y0 + 1
                        vx0 = ((x0 >= 0) & (x0 <= wl - 1)).astype(jnp.float32)
                        vx1 = ((x1 >= 0) & (x1 <= wl - 1)).astype(jnp.float32)
                        vy0 = ((y0 >= 0) & (y0 <= hl - 1)).astype(jnp.float32)
                        vy1 = ((y1 >= 0) & (y1 <= hl - 1)).astype(jnp.float32)
                        wx0 = (1.0 - fx) * vx0
                        wx1 = fx * vx1
                        wy0 = (1.0 - fy) * a * vy0
                        wy1 = fy * a * vy1
                        zero = jnp.zeros((16,), jnp.float32)
                        xlo = x0 < 0
                        xhi = x0 >= wl - 1
                        bx0 = jnp.where(xlo, wx1, jnp.where(xhi, zero, wx0))
                        bx1 = jnp.where(xlo, zero, jnp.where(xhi, wx0, wx1))
                        ylo = y0 < 0
                        yhi = y0 >= hl - 1
                        by0 = jnp.where(ylo, wy1, jnp.where(yhi, zero, wy0))
                        by1 = jnp.where(ylo, zero, jnp.where(yhi, wy0, wy1))
                        bxc = jnp.clip(x0, 0, wl - 2)
                        byc = jnp.clip(y0, 0, hl - 2)
                        rb = (byc * wl + base + bxc) * HD
                        w00 = bx0 * by0
                        w01 = bx1 * by0
                        w10 = bx0 * by1
                        w11 = bx1 * by1
                        dy = wl * HD
                        newacc = []
                        for q in range(16):
                            rq = rb[q]
                            v00 = slab_v[pl.ds(rq, HD)]
                            v01 = slab_v[pl.ds(rq + HD, HD)]
                            v10 = slab_v[pl.ds(rq + dy, HD)]
                            v11 = slab_v[pl.ds(rq + (dy + HD), HD)]
                            newacc.append(acc_c[q]
                                          + (v00 * w00[q] + v01 * w01[q])
                                          + (v10 * w10[q] + v11 * w11[q]))
                        return tuple(newacc)

                    acc = plsc.parallel_loop(
                        0, POINTS, unroll=8, carry=tuple(acc))(p_body)
                for q in range(16):
                    out_v[buf, qoff + q, :] = acc[q]
            out_copy(scn).start()
            return 0

        lax.fori_loop(0, NCH, chunk_body, 0)
        for k in (NCH - 2, NCH - 1):
            out_copy(k).wait()
        return 0

    lax.fori_loop(0, 3, unit_body, 0)


def _sc_sample(value_flat, so_arr, aw_arr, coor_arr):
    mesh = plsc.VectorSubcoreMesh(core_axis_name="c", subcore_axis_name="s")
    fn = functools.partial(
        pl.kernel,
        mesh=mesh,
        compiler_params=pltpu.CompilerParams(needs_layout_passes=False),
        out_type=jax.ShapeDtypeStruct((NC, HEADS, NP, HD), jnp.float32),
        scratch_types=[
            pltpu.VMEM((SLAB,), jnp.float32),
            pltpu.VMEM((2, LP, CHUNK), jnp.float32),
            pltpu.VMEM((2, LP, CHUNK), jnp.float32),
            pltpu.VMEM((2, LP, CHUNK), jnp.float32),
            pltpu.VMEM((2, CHUNK), jnp.float32),
            pltpu.VMEM((2, CHUNK), jnp.float32),
            pltpu.VMEM((2, CHUNK, HD), jnp.float32),
            pltpu.SemaphoreType.DMA,
            pltpu.SemaphoreType.DMA,
        ],
    )(_sc_sample_body)
    return fn(value_flat, so_arr, aw_arr, coor_arr)


_CB = 256


def _finish_kernel(v_ref, m_ref, ic_ref, f_ref, wout_ref, par_ref, o_ref):
    s = jnp.zeros((_CB, C), jnp.float32)
    for cam in range(NC):
        s = s + v_ref[cam] * m_ref[:, cam:cam + 1]
    s = s * ic_ref[:, 0:1]
    xo = lax.dot_general(s, wout_ref[...], (((1,), (1,)), ((), ())),
                         preferred_element_type=jnp.float32)
    xo = xo + par_ref[0:1, :] + f_ref[...]
    mu = jnp.mean(xo, axis=1, keepdims=True)
    d = xo - mu
    var = jnp.mean(d * d, axis=1, keepdims=True)
    o_ref[...] = d * lax.rsqrt(var + 1e-5) * par_ref[1:2, :] + par_ref[2:3, :]


def _finish(outsc, maskT, invcT, featP, W_out, params):
    return pl.pallas_call(
        _finish_kernel,
        grid=(NP // _CB,),
        in_specs=[
            pl.BlockSpec((NC, _CB, C), lambda i: (0, i, 0)),
            pl.BlockSpec((_CB, 8), lambda i: (i, 0)),
            pl.BlockSpec((_CB, 8), lambda i: (i, 0)),
            pl.BlockSpec((_CB, C), lambda i: (i, 0)),
            pl.BlockSpec((C, C), lambda i: (0, 0)),
            pl.BlockSpec((8, C), lambda i: (0, 0)),
        ],
        out_specs=pl.BlockSpec((_CB, C), lambda i: (i, 0)),
        out_shape=jax.ShapeDtypeStruct((NP, C), jnp.float32),
    )(outsc, maskT, invcT, featP, W_out, params)


def kernel(means, feature, feat0, feat1, feat2, cam2ego_lidar, intrins,
           post_rots, post_trans, W_off, b_off, W_attn, b_attn, W_val, b_val,
           W_out, b_out, cams_embeds, level_embeds, ln_w, ln_b, H, W):
    f32 = jnp.float32
    means = means.astype(f32)
    feature2d = feature[0].astype(f32)
    Bv, Nq, _ = means.shape

    inv = jnp.linalg.inv(cam2ego_lidar)
    mh = jnp.concatenate([means, jnp.ones((Bv, Nq, 1), f32)], -1)
    means_cam = (inv[:, :, None] @ mh[:, None, :, :, None])[..., :3, :]
    means_img = (intrins[:, :, None] @ means_cam)[..., 0]
    mn = jnp.concatenate(
        [means_img[..., :2] / (means_img[..., 2:] + 0.0001),
         means_img[..., 2:]], -1)
    mn = (post_rots[:, :, None] @ mn[..., None])[..., 0] + post_trans[:, :, None]
    coor = mn[..., :2] / jnp.array([W, H], f32)
    depth = mn[..., 2]
    mask = ((depth > 0.01)
            & (coor[..., 0] > 0.0) & (coor[..., 0] < 1.0)
            & (coor[..., 1] > 0.0) & (coor[..., 1] < 1.0))

    ffT = jnp.concatenate(
        [feat0[0].reshape(NC, C, -1).transpose(0, 2, 1),
         feat1[0].reshape(NC, C, -1).transpose(0, 2, 1),
         feat2[0].reshape(NC, C, -1).transpose(0, 2, 1)], 1)

    featP = jnp.pad(feature2d, ((0, NP - Nq), (0, 0)))
    featT = featP.T

    W_off2 = W_off.reshape(HEADS, LEVELS, POINTS, 2, C).transpose(
        3, 0, 1, 2, 4).reshape(2 * HEADS * LP, C)
    b_off2 = b_off.reshape(HEADS, LEVELS, POINTS, 2).transpose(
        3, 0, 1, 2).reshape(2 * HEADS * LP)

    coorT = coor[0].transpose(2, 0, 1)
    coorP = jnp.pad(coorT, ((0, 0), (0, 0), (0, NP - Nq)),
                    constant_values=0.5).reshape(2 * NC * NP)

    mf = mask[0].astype(f32)
    cnt = jnp.clip(mf.sum(0), 1.0)
    maskT = jnp.pad(mf.T, ((0, NP - Nq), (0, 8 - NC)))
    invcT = jnp.broadcast_to((1.0 / jnp.pad(cnt, (0, NP - Nq),
                                            constant_values=1.0))[:, None],
                             (NP, 8))

    value = _project_value(ffT, W_val, b_val, cams_embeds, level_embeds)
    so_arr, aw_arr = _off_attn(featT, W_off2, b_off2, W_attn, b_attn)

    value_flat = value.reshape(NC * HEADS * SLAB)
    outsc = _sc_sample(value_flat, so_arr, aw_arr, coorP)

    outsc_q = outsc.transpose(0, 2, 1, 3).reshape(NC, NP, C)

    params = jnp.pad(jnp.stack([b_out, ln_w, ln_b], 0), ((0, 5), (0, 0)))
    y = _finish(outsc_q, maskT, invcT, featP, W_out, params)
    return y[:Nq].reshape(1, Nq, C)


# speedup vs baseline: 1.7651x; 1.7651x over previous
import functools

import jax
import jax.numpy as jnp
from jax import lax
from jax.experimental import pallas as pl
from jax.experimental.pallas import tpu as pltpu
from jax.experimental.pallas import tpu_sc as plsc

N, NC, C = 2500, 6, 128
HEADS, LEVELS, POINTS = 8, 3, 8
LP = LEVELS * POINTS
NP = 2560
HD = C // HEADS
SPATIAL = ((32, 88), (16, 44), (8, 22))
LVL_BASE = (0, 2816, 3520)
PIX = 3696
SLAB = PIX * HD

QHALF = NP // 2
CHUNK = 128
NGROUP = CHUNK // 16
NCH = QHALF // CHUNK


def _value_kernel(ff_ref, wval_ref, bval_ref, cam_ref, lvl_ref, out_ref):
    ff = ff_ref[0]
    res = lax.dot_general(ff, wval_ref[...], (((1,), (1,)), ((), ())),
                          preferred_element_type=jnp.float32)
    res = res + bval_ref[0:1, :]
    emb = cam_ref[0] + lvl_ref[...]
    bias3 = lax.dot_general(emb, wval_ref[...], (((1,), (1,)), ((), ())),
                            preferred_element_type=jnp.float32)
    for h in range(HEADS):
        c0 = h * HD
        for l in range(LEVELS):
            r0 = LVL_BASE[l]
            r1 = LVL_BASE[l + 1] if l + 1 < LEVELS else PIX
            out_ref[0, h, r0:r1, :] = (res[r0:r1, c0:c0 + HD]
                                       + bias3[l:l + 1, c0:c0 + HD])


def _project_value(ffT, W_val, b_val, cams_embeds, level_embeds):
    bval_m = jnp.broadcast_to(b_val[None, :], (8, C))
    return pl.pallas_call(
        _value_kernel,
        grid=(NC,),
        in_specs=[
            pl.BlockSpec((1, PIX, C), lambda i: (i, 0, 0)),
            pl.BlockSpec((C, C), lambda i: (0, 0)),
            pl.BlockSpec((8, C), lambda i: (0, 0)),
            pl.BlockSpec((1, 1, C), lambda i: (i, 0, 0)),
            pl.BlockSpec((LEVELS, C), lambda i: (0, 0)),
        ],
        out_specs=pl.BlockSpec((1, HEADS, PIX, HD), lambda i: (i, 0, 0, 0)),
        out_shape=jax.ShapeDtypeStruct((NC, HEADS, PIX, HD), jnp.float32),
    )(ffT, W_val, bval_m, cams_embeds[:, None, :], level_embeds)


_QB = 512


def _offattn_kernel(ft_ref, woff_ref, boff_ref, wattn_ref, battn_ref,
                    so_ref, aw_ref):
    ft = ft_ref[...]
    lo = lax.dot_general(woff_ref[...], ft, (((1,), (0,)), ((), ())),
                         preferred_element_type=jnp.float32)
    lo = lo + boff_ref[:, 0:1]
    so_ref[...] = lo.reshape(2, HEADS * LP, _QB)
    la = lax.dot_general(wattn_ref[...], ft, (((1,), (0,)), ((), ())),
                         preferred_element_type=jnp.float32)
    la = la + battn_ref[:, 0:1]
    for h in range(HEADS):
        s = la[h * LP:(h + 1) * LP, :]
        m = jnp.max(s, axis=0, keepdims=True)
        e = jnp.exp(s - m)
        aw_ref[h * LP:(h + 1) * LP, :] = e / jnp.sum(e, axis=0, keepdims=True)


def _off_attn(featT, W_off2, b_off2, W_attn, b_attn):
    boff_m = jnp.broadcast_to(b_off2[:, None], (2 * HEADS * LP, 8))
    battn_m = jnp.broadcast_to(b_attn[:, None], (HEADS * LP, 8))
    return pl.pallas_call(
        _offattn_kernel,
        grid=(NP // _QB,),
        in_specs=[
            pl.BlockSpec((C, _QB), lambda i: (0, i)),
            pl.BlockSpec((2 * HEADS * LP, C), lambda i: (0, 0)),
            pl.BlockSpec((2 * HEADS * LP, 8), lambda i: (0, 0)),
            pl.BlockSpec((HEADS * LP, C), lambda i: (0, 0)),
            pl.BlockSpec((HEADS * LP, 8), lambda i: (0, 0)),
        ],
        out_specs=[
            pl.BlockSpec((2, HEADS * LP, _QB), lambda i: (0, 0, i)),
            pl.BlockSpec((HEADS * LP, _QB), lambda i: (0, i)),
        ],
        out_shape=[
            jax.ShapeDtypeStruct((2, HEADS * LP, NP), jnp.float32),
            jax.ShapeDtypeStruct((HEADS * LP, NP), jnp.float32),
        ],
    )(featT, W_off2, boff_m, W_attn, battn_m)


def _sc_sample_body(value_hbm, so_hbm, aw_hbm, coor_hbm, out_hbm,
                    slab_v, sox_v, soy_v, aw_v, cx_v, cy_v, out_v,
                    insem, outsem):
    info = plsc.get_sparse_core_info()
    ncores = info.num_cores
    wid = lax.axis_index("s") * ncores + lax.axis_index("c")

    def unit_body(j, _):
        u = wid * 3 + j
        cam = u // 16
        r = u - cam * 16
        h = r // 2
        qh = r - h * 2
        pltpu.sync_copy(value_hbm.at[pl.ds((cam * HEADS + h) * SLAB, SLAB)],
                        slab_v)
        qs = qh * QHALF

        def in_copies(scn):
            buf = scn & 1
            qb = qs + scn * CHUNK
            return (
                pltpu.make_async_copy(
                    so_hbm.at[0, pl.ds(h * LP, LP), pl.ds(qb, CHUNK)],
                    sox_v.at[buf], insem),
                pltpu.make_async_copy(
                    so_hbm.at[1, pl.ds(h * LP, LP), pl.ds(qb, CHUNK)],
                    soy_v.at[buf], insem),
                pltpu.make_async_copy(
                    aw_hbm.at[pl.ds(h * LP, LP), pl.ds(qb, CHUNK)],
                    aw_v.at[buf], insem),
                pltpu.make_async_copy(
                    coor_hbm.at[pl.ds(cam * NP + qb, CHUNK)],
                    cx_v.at[buf], insem),
                pltpu.make_async_copy(
                    coor_hbm.at[pl.ds((NC + cam) * NP + qb, CHUNK)],
                    cy_v.at[buf], insem),
            )

        def out_copy(scn):
            buf = scn & 1
            qb = qs + scn * CHUNK
            return pltpu.make_async_copy(
                out_v.at[buf],
                out_hbm.at[cam, h, pl.ds(qb, CHUNK), :], outsem)

        for c in in_copies(0):
            c.start()

        def chunk_body(scn, _2):
            buf = scn & 1
            qb = qs + scn * CHUNK
            for c in in_copies(scn):
                c.wait()

            @pl.when(scn + 1 < NCH)
            def _fire_next():
                for c in in_copies(scn + 1):
                    c.start()

            @pl.when(scn >= 2)
            def _reclaim_out():
                out_copy(scn - 2).wait()

            @plsc.parallel_loop(0, NGROUP)
            def group_body(g):
                qoff = g * 16
                cx = cx_v[buf, pl.ds(qoff, 16)]
                cy = cy_v[buf, pl.ds(qoff, 16)]
                acc = [jnp.zeros((16,), jnp.float32) for _ in range(16)]
                for l in range(LEVELS):
                    hl, wl = SPATIAL[l]
                    base = LVL_BASE[l]
                    xb = cx * float(wl) - 0.5
                    yb = cy * float(hl) - 0.5

                    def p_body(p, acc_c, l=l, hl=hl, wl=wl, base=base,
                               xb=xb, yb=yb):
                        lp = l * POINTS + p
                        sx = sox_v[buf, lp, pl.ds(qoff, 16)]
                        sy = soy_v[buf, lp, pl.ds(qoff, 16)]
                        a = aw_v[buf, lp, pl.ds(qoff, 16)]
                        x = xb + sx
                        y = yb + sy
                        xt = x.astype(jnp.int32)
                        xtf = xt.astype(jnp.float32)
                        x0f = jnp.where(xtf > x, xtf - 1.0, xtf)
                        x0 = x0f.astype(jnp.int32)
                        yt = y.astype(jnp.int32)
                        ytf = yt.astype(jnp.float32)
                        y0f = jnp.where(ytf > y, ytf - 1.0, ytf)
                        y0 = y0f.astype(jnp.int32)
                        fx = x - x0f
                        fy = y - y0f
                        x1 = x0 + 1
                        y1 = y0 + 1
                        vx0 = ((x0 >= 0) & (x0 <= wl - 1)).astype(jnp.float32)
                        vx1 = ((x1 >= 0) & (x1 <= wl - 1)).astype(jnp.float32)
                        vy0 = ((y0 >= 0) & (y0 <= hl - 1)).astype(jnp.float32)
                        vy1 = ((y1 >= 0) & (y1 <= hl - 1)).astype(jnp.float32)
                        wx0 = (1.0 - fx) * vx0
                        wx1 = fx * vx1
                        wy0 = (1.0 - fy) * a * vy0
                        wy1 = fy * a * vy1
                        zero = jnp.zeros((16,), jnp.float32)
                        xlo = x0 < 0
                        xhi = x0 >= wl - 1
                        bx0 = jnp.where(xlo, wx1, jnp.where(xhi, zero, wx0))
                        bx1 = jnp.where(xlo, zero, jnp.where(xhi, wx0, wx1))
                        ylo = y0 < 0
                        yhi = y0 >= hl - 1
                        by0 = jnp.where(ylo, wy1, jnp.where(yhi, zero, wy0))
                        by1 = jnp.where(ylo, zero, jnp.where(yhi, wy0, wy1))
                        bxc = jnp.clip(x0, 0, wl - 2)
                        byc = jnp.clip(y0, 0, hl - 2)
                        rb = (byc * wl + base + bxc) * HD
                        w00 = bx0 * by0
                        w01 = bx1 * by0
                        w10 = bx0 * by1
                        w11 = bx1 * by1
                        dy = wl * HD
                        newacc = []
                        for q in range(16):
                            rq = rb[q]
                            v00 = slab_v[pl.ds(rq, HD)]
                            v01 = slab_v[pl.ds(rq + HD, HD)]
                            v10 = slab_v[pl.ds(rq + dy, HD)]
                            v11 = slab_v[pl.ds(rq + (dy + HD), HD)]
                            newacc.append(acc_c[q]
                                          + (v00 * w00[q] + v01 * w01[q])
                                          + (v10 * w10[q] + v11 * w11[q]))
                        return tuple(newacc)

                    acc = plsc.parallel_loop(
                        0, POINTS, unroll=4, carry=tuple(acc))(p_body)
                for q in range(16):
                    out_v[buf, qoff + q, :] = acc[q]
            out_copy(scn).start()
            return 0

        lax.fori_loop(0, NCH, chunk_body, 0)
        for k in (NCH - 2, NCH - 1):
            out_copy(k).wait()
        return 0

    lax.fori_loop(0, 3, unit_body, 0)


def _sc_sample(value_flat, so_arr, aw_arr, coor_arr):
    mesh = plsc.VectorSubcoreMesh(core_axis_name="c", subcore_axis_name="s")
    fn = functools.partial(
        pl.kernel,
        mesh=mesh,
        compiler_params=pltpu.CompilerParams(needs_layout_passes=False),
        out_type=jax.ShapeDtypeStruct((NC, HEADS, NP, HD), jnp.float32),
        scratch_types=[
            pltpu.VMEM((SLAB,), jnp.float32),
            pltpu.VMEM((2, LP, CHUNK), jnp.float32),
            pltpu.VMEM((2, LP, CHUNK), jnp.float32),
            pltpu.VMEM((2, LP, CHUNK), jnp.float32),
            pltpu.VMEM((2, CHUNK), jnp.float32),
            pltpu.VMEM((2, CHUNK), jnp.float32),
            pltpu.VMEM((2, CHUNK, HD), jnp.float32),
            pltpu.SemaphoreType.DMA,
            pltpu.SemaphoreType.DMA,
        ],
    )(_sc_sample_body)
    return fn(value_flat, so_arr, aw_arr, coor_arr)


_CB = 256


def _finish_kernel(v_ref, m_ref, ic_ref, f_ref, wout_ref, par_ref, o_ref):
    s = jnp.zeros((_CB, C), jnp.float32)
    for cam in range(NC):
        s = s + v_ref[cam] * m_ref[:, cam:cam + 1]
    s = s * ic_ref[:, 0:1]
    xo = lax.dot_general(s, wout_ref[...], (((1,), (1,)), ((), ())),
                         preferred_element_type=jnp.float32)
    xo = xo + par_ref[0:1, :] + f_ref[...]
    mu = jnp.mean(xo, axis=1, keepdims=True)
    d = xo - mu
    var = jnp.mean(d * d, axis=1, keepdims=True)
    o_ref[...] = d * lax.rsqrt(var + 1e-5) * par_ref[1:2, :] + par_ref[2:3, :]


def _finish(outsc, maskT, invcT, featP, W_out, params):
    return pl.pallas_call(
        _finish_kernel,
        grid=(NP // _CB,),
        in_specs=[
            pl.BlockSpec((NC, _CB, C), lambda i: (0, i, 0)),
            pl.BlockSpec((_CB, 8), lambda i: (i, 0)),
            pl.BlockSpec((_CB, 8), lambda i: (i, 0)),
            pl.BlockSpec((_CB, C), lambda i: (i, 0)),
            pl.BlockSpec((C, C), lambda i: (0, 0)),
            pl.BlockSpec((8, C), lambda i: (0, 0)),
        ],
        out_specs=pl.BlockSpec((_CB, C), lambda i: (i, 0)),
        out_shape=jax.ShapeDtypeStruct((NP, C), jnp.float32),
    )(outsc, maskT, invcT, featP, W_out, params)


def kernel(means, feature, feat0, feat1, feat2, cam2ego_lidar, intrins,
           post_rots, post_trans, W_off, b_off, W_attn, b_attn, W_val, b_val,
           W_out, b_out, cams_embeds, level_embeds, ln_w, ln_b, H, W):
    f32 = jnp.float32
    means = means.astype(f32)
    feature2d = feature[0].astype(f32)
    Bv, Nq, _ = means.shape

    inv = jnp.linalg.inv(cam2ego_lidar)
    mh = jnp.concatenate([means, jnp.ones((Bv, Nq, 1), f32)], -1)
    means_cam = (inv[:, :, None] @ mh[:, None, :, :, None])[..., :3, :]
    means_img = (intrins[:, :, None] @ means_cam)[..., 0]
    mn = jnp.concatenate(
        [means_img[..., :2] / (means_img[..., 2:] + 0.0001),
         means_img[..., 2:]], -1)
    mn = (post_rots[:, :, None] @ mn[..., None])[..., 0] + post_trans[:, :, None]
    coor = mn[..., :2] / jnp.array([W, H], f32)
    depth = mn[..., 2]
    mask = ((depth > 0.01)
            & (coor[..., 0] > 0.0) & (coor[..., 0] < 1.0)
            & (coor[..., 1] > 0.0) & (coor[..., 1] < 1.0))

    ffT = jnp.concatenate(
        [feat0[0].reshape(NC, C, -1).transpose(0, 2, 1),
         feat1[0].reshape(NC, C, -1).transpose(0, 2, 1),
         feat2[0].reshape(NC, C, -1).transpose(0, 2, 1)], 1)

    featP = jnp.pad(feature2d, ((0, NP - Nq), (0, 0)))
    featT = featP.T

    W_off2 = W_off.reshape(HEADS, LEVELS, POINTS, 2, C).transpose(
        3, 0, 1, 2, 4).reshape(2 * HEADS * LP, C)
    b_off2 = b_off.reshape(HEADS, LEVELS, POINTS, 2).transpose(
        3, 0, 1, 2).reshape(2 * HEADS * LP)

    coorT = coor[0].transpose(2, 0, 1)
    coorP = jnp.pad(coorT, ((0, 0), (0, 0), (0, NP - Nq)),
                    constant_values=0.5).reshape(2 * NC * NP)

    mf = mask[0].astype(f32)
    cnt = jnp.clip(mf.sum(0), 1.0)
    maskT = jnp.pad(mf.T, ((0, NP - Nq), (0, 8 - NC)))
    invcT = jnp.broadcast_to((1.0 / jnp.pad(cnt, (0, NP - Nq),
                                            constant_values=1.0))[:, None],
                             (NP, 8))

    value = _project_value(ffT, W_val, b_val, cams_embeds, level_embeds)
    so_arr, aw_arr = _off_attn(featT, W_off2, b_off2, W_attn, b_attn)

    value_flat = value.reshape(NC * HEADS * SLAB)
    outsc = _sc_sample(value_flat, so_arr, aw_arr, coorP)

    outsc_q = outsc.transpose(0, 2, 1, 3).reshape(NC, NP, C)

    params = jnp.pad(jnp.stack([b_out, ln_w, ln_b], 0), ((0, 5), (0, 0)))
    y = _finish(outsc_q, maskT, invcT, featP, W_out, params)
    return y[:Nq].reshape(1, Nq, C)
